# baseline (speedup 1.0000x reference)
import jax
import jax.numpy as jnp
from jax import lax
from jax.experimental import pallas as pl
from jax.experimental.pallas import tpu as pltpu
from jax.experimental.pallas import tpu_sc as plsc

NN = 10000
NE = 10000
KK = 160000
FF = 128
HID = 32
DD = 4

NC = 2
NS = 16
NWORK = NC * NS
NPAD = 10112
HNP = NPAD // 2
ACCR = 5120
RPTA = ACCR // NS
KW = KK // NWORK
CH = 125
CHA = CH * 16
CW = 128
NCHUNK = KW // CH
NCHALL = NWORK * NCHUNK
NCHSC = NCHALL // NS

_f32 = jnp.float32


def _dense_pre_body(x_ref, he_ref, wlin_ref, blin_ref, w1b_ref, bw1b_ref,
                    wa_ref, wb_ref, bs_ref, hw1_ref, a128_ref, b128_ref):
    h = jnp.dot(x_ref[...], wlin_ref[...], preferred_element_type=_f32)
    h = h + blin_ref[...]
    hw1_ref[...] = jnp.dot(h, w1b_ref[...], preferred_element_type=_f32) + bw1b_ref[...]
    a128_ref[...] = jnp.dot(h, wa_ref[...], preferred_element_type=_f32) + bs_ref[...]
    heh = jnp.dot(he_ref[...], wlin_ref[...], preferred_element_type=_f32)
    heh = heh + blin_ref[...]
    b128_ref[...] = jnp.dot(heh, wb_ref[...], preferred_element_type=_f32)


def _inv_body(dp_ref, bp_ref, dinv_ref, binv_ref):
    for c in (0, 1):
        sl = slice(c * HNP, (c + 1) * HNP)
        ds = dp_ref[c, 0:HNP]
        dinv_ref[sl, :] = jnp.where(ds > 0.0, 1.0 / ds, 0.0)
        bs = bp_ref[c, 0:HNP]
        binv_ref[sl, :] = jnp.where(bs > 0.0, 1.0 / bs, 0.0)


def _combine_binv_body(mp_ref, binv_ref, m_ref):
    for c in (0, 1):
        sl = slice(c * HNP, (c + 1) * HNP)
        m_ref[sl, :] = mp_ref[c, 0:HNP] * binv_ref[sl, :]


def _mid_body(op_ref, dinv_ref, b1_ref, w2b_ref, bw2b_ref, hw2_ref):
    o = jnp.concatenate([op_ref[0, 0:HNP], op_ref[1, 0:HNP]], axis=0)
    z = o * dinv_ref[...] + b1_ref[...]
    h2 = jnp.where(z > 0.0, z, jnp.exp(z) - 1.0)
    hw2_ref[...] = jnp.dot(h2, w2b_ref[...], preferred_element_type=_f32) + bw2b_ref[...]


def _final_body(op_ref, dinv_ref, b2_ref, out_ref):
    o = jnp.concatenate([op_ref[0, 0:HNP], op_ref[1, 0:HNP]], axis=0)
    z = o * dinv_ref[...] + b2_ref[...]
    out_ref[...] = jnp.where(z > 0.0, z, jnp.exp(z) - 1.0)


def _expand_alpha(al_v, i):
    s = al_v[pl.ds(i * 16, 16)]
    out = []
    for d in range(DD):
        scale = s.at[jnp.full((16,), d, jnp.int32)].get(mode="promise_in_bounds")
        out.append(scale)
    return out


UNROLL = 5


def _scale_mul_chunk(buf_b, al_b):
    def body(t, c):
        for u in range(UNROLL):
            i = t * UNROLL + u
            sc = _expand_alpha(al_b, i)
            for d in range(DD):
                lo = d * HID
                buf_b[i, lo:lo + 16] = buf_b[i, lo:lo + 16] * sc[d]
                buf_b[i, lo + 16:lo + 32] = buf_b[i, lo + 16:lo + 32] * sc[d]
        return c
    lax.fori_loop(0, CH // UNROLL, body, 0)


def _scale_set_chunk(buf_b, al_b):
    def body(t, c):
        for u in range(UNROLL):
            i = t * UNROLL + u
            sc = _expand_alpha(al_b, i)
            for d in range(DD):
                lo = d * HID
                buf_b[i, lo:lo + 16] = buf_b[i, lo:lo + 16] * 0.0 + sc[d]
                buf_b[i, lo + 16:lo + 32] = buf_b[i, lo + 16:lo + 32] * 0.0 + sc[d]
        return c
    lax.fori_loop(0, CH // UNROLL, body, 0)


def _alpha_body(a128_hbm, b128_hbm, row_hbm, col_hbm,
                alpha_hbm,
                ridx0_v, ridx1_v, cidx0_v, cidx1_v, arow0_v, arow1_v,
                brow0_v, brow1_v, al_v, sema0, sema1, semb0, semb1):
    cid = lax.axis_index("c")
    sid = lax.axis_index("s")
    w = sid * NC + cid
    ridx = (ridx0_v, ridx1_v)
    cidx = (cidx0_v, cidx1_v)
    arow = (arow0_v, arow1_v)
    brow = (brow0_v, brow1_v)
    semas = (sema0, sema1)
    sembs = (semb0, semb1)

    def fetch(j, b):
        r = w * NCHUNK + j
        pltpu.sync_copy(row_hbm.at[r], ridx[b])
        pltpu.sync_copy(col_hbm.at[r], cidx[b])
        pltpu.async_copy(a128_hbm.at[ridx[b]], arow[b], semas[b])
        pltpu.async_copy(b128_hbm.at[cidx[b]], brow[b], sembs[b])

    fetch(0, 0)

    def pair(t, carry):
        for b in (0, 1):
            j = t * 2 + b

            @pl.when(j + 1 < NCHUNK)
            def _(j=j, b=b):
                fetch(j + 1, 1 - b)

            pltpu.make_async_copy(a128_hbm.at[ridx[b]], arow[b], semas[b]).wait()
            pltpu.make_async_copy(b128_hbm.at[cidx[b]], brow[b], sembs[b]).wait()
            ar = arow[b]
            br = brow[b]

            def edge(t2, c2, ar=ar, br=br):
                for u in range(UNROLL):
                    i = t2 * UNROLL + u
                    v = ar[i, 0:16] + br[i, 0:16]
                    al_v[pl.ds(i * 16, 16)] = 1.0 / (1.0 + jnp.exp(-v))
                return c2

            lax.fori_loop(0, CH // UNROLL, edge, 0)
            r = w * NCHUNK + j
            pltpu.sync_copy(al_v, alpha_hbm.at[pl.ds(r * CHA, CHA)])
        return carry

    lax.fori_loop(0, NCHUNK // 2, pair, 0)


def _deg_body(alpha_hbm, sidx_hbm, z128_hbm, out_hbm,
              idx_v, al_v, buf_v, acc_s, sem):
    del sem
    cid = lax.axis_index("c")
    sid = lax.axis_index("s")
    pltpu.sync_copy(z128_hbm.at[pl.ds(sid * RPTA, RPTA)],
                    acc_s.at[pl.ds(sid * RPTA, RPTA)])
    pltpu.sync_copy(z128_hbm.at[pl.ds(0, CW)], buf_v)
    plsc.subcore_barrier()

    def chunk(j, carry):
        r = sid * NCHSC + j
        pltpu.sync_copy(sidx_hbm.at[cid, r], idx_v)
        pltpu.sync_copy(alpha_hbm.at[pl.ds(r * CHA, CHA)], al_v)

        _scale_set_chunk(buf_v, al_v)
        pltpu.sync_copy(buf_v, acc_s.at[idx_v], add=True)
        return carry

    lax.fori_loop(0, NCHSC, chunk, 0)
    plsc.subcore_barrier()
    pltpu.sync_copy(acc_s.at[pl.ds(sid * RPTA, RPTA)],
                    out_hbm.at[cid, pl.ds(sid * RPTA, RPTA)])


def _stage_body(table_hbm, alpha_hbm, gidx_hbm, sidx_hbm, z128_hbm,
                out_hbm,
                gidx0_v, gidx1_v, sidx0_v, sidx1_v, al0_v, al1_v,
                buf0_v, buf1_v, acc_s, sem0, sem1):
    cid = lax.axis_index("c")
    sid = lax.axis_index("s")
    pltpu.sync_copy(z128_hbm.at[pl.ds(sid * RPTA, RPTA)],
                    acc_s.at[pl.ds(sid * RPTA, RPTA)])
    plsc.subcore_barrier()
    gidx = (gidx0_v, gidx1_v)
    sidx = (sidx0_v, sidx1_v)
    al = (al0_v, al1_v)
    buf = (buf0_v, buf1_v)
    sems = (sem0, sem1)

    def fetch(j, b):
        r = sid * NCHSC + j
        pltpu.sync_copy(gidx_hbm.at[r], gidx[b])
        pltpu.sync_copy(sidx_hbm.at[cid, r], sidx[b])
        pltpu.sync_copy(alpha_hbm.at[pl.ds(r * CHA, CHA)], al[b])
        pltpu.async_copy(table_hbm.at[gidx[b]], buf[b], sems[b])

    fetch(0, 0)

    def pair(t, carry):
        for b in (0, 1):
            j = t * 2 + b

            @pl.when(j + 1 < NCHSC)
            def _(j=j, b=b):
                fetch(j + 1, 1 - b)

            pltpu.make_async_copy(table_hbm.at[gidx[b]], buf[b], sems[b]).wait()
            _scale_mul_chunk(buf[b], al[b])
            pltpu.sync_copy(buf[b], acc_s.at[sidx[b]], add=True)
        return carry

    lax.fori_loop(0, NCHSC // 2, pair, 0)
    plsc.subcore_barrier()
    pltpu.sync_copy(acc_s.at[pl.ds(sid * RPTA, RPTA)],
                    out_hbm.at[cid, pl.ds(sid * RPTA, RPTA)])


def _tc_call(body, out_shapes):
    return pl.pallas_call(body, out_shape=out_shapes)


def kernel(x, edge_index, node_types, hyperedge_types, hyperedge_attr,
           W_lin, b_lin, W_sheaf, b_sheaf, W1, bW1, bias1, W2, bW2, bias2):
    del node_types, hyperedge_types

    x_pad = jnp.pad(x, ((0, NPAD - NN), (0, 0)))
    he_pad = jnp.pad(hyperedge_attr, ((0, NPAD - NE), (0, 0)))
    pad_idx = NN + (jnp.arange(NCHALL * (CW - CH), dtype=jnp.int32)
                    .reshape(NCHALL, CW - CH) % (NPAD - NN))
    row = jnp.concatenate(
        [edge_index[0].astype(jnp.int32).reshape(NCHALL, CH), pad_idx], axis=1)
    col = jnp.concatenate(
        [edge_index[1].astype(jnp.int32).reshape(NCHALL, CH), pad_idx], axis=1)

    trash = HNP + (jnp.arange(CW, dtype=jnp.int32)[None, :] % (ACCR - HNP))

    def _split(g):
        l0 = jnp.where(g < HNP, g, trash)
        l1 = jnp.where(g >= HNP, g - HNP, trash)
        return jnp.stack([l0, l1])

    row_l = _split(row)
    col_l = _split(col)

    eyeh = jnp.eye(HID, dtype=_f32)
    m_mean = jnp.tile(eyeh, (DD, 1)) / DD
    wa = jnp.pad(m_mean @ W_sheaf[:HID], ((0, 0), (0, FF - DD)))
    wb = jnp.pad(m_mean @ W_sheaf[HID:], ((0, 0), (0, FF - DD)))
    bs128 = jnp.pad(b_sheaf, (0, FF - DD)).reshape(1, FF)
    w1b = jnp.kron(jnp.eye(DD, dtype=_f32), W1)
    w2b = jnp.kron(jnp.eye(DD, dtype=_f32), W2)
    bw1b = jnp.tile(bW1, DD).reshape(1, FF)
    bw2b = jnp.tile(bW2, DD).reshape(1, FF)
    b1t = jnp.tile(bias1, DD).reshape(1, FF)
    b2t = jnp.tile(bias2, DD).reshape(1, FF)
    z128 = jnp.zeros((NPAD, FF), _f32)

    hw1, a128, b128 = _tc_call(_dense_pre_body, (
        jax.ShapeDtypeStruct((NPAD, FF), _f32),
        jax.ShapeDtypeStruct((NPAD, FF), _f32),
        jax.ShapeDtypeStruct((NPAD, FF), _f32),
    ))(x_pad, he_pad, W_lin, b_lin.reshape(1, FF), w1b, bw1b, wa, wb, bs128)

    mesh = plsc.VectorSubcoreMesh(core_axis_name="c", subcore_axis_name="s")

    alpha = pl.kernel(
        _alpha_body,
        out_type=jax.ShapeDtypeStruct((NCHALL * CHA,), _f32),
        mesh=mesh,
        scratch_types=[
            pltpu.VMEM((CW,), jnp.int32),
            pltpu.VMEM((CW,), jnp.int32),
            pltpu.VMEM((CW,), jnp.int32),
            pltpu.VMEM((CW,), jnp.int32),
            pltpu.VMEM((CW, FF), _f32),
            pltpu.VMEM((CW, FF), _f32),
            pltpu.VMEM((CW, FF), _f32),
            pltpu.VMEM((CW, FF), _f32),
            pltpu.VMEM((CHA,), _f32),
            pltpu.SemaphoreType.DMA,
            pltpu.SemaphoreType.DMA,
            pltpu.SemaphoreType.DMA,
            pltpu.SemaphoreType.DMA,
        ],
    )(a128, b128, row, col)

    deg = pl.kernel(
        _deg_body,
        out_type=jax.ShapeDtypeStruct((NC, ACCR, FF), _f32),
        mesh=mesh,
        scratch_types=[
            pltpu.VMEM((CW,), jnp.int32),
            pltpu.VMEM((CHA,), _f32),
            pltpu.VMEM((CW, FF), _f32),
            pltpu.VMEM_SHARED((ACCR, FF), _f32),
            pltpu.SemaphoreType.DMA,
        ],
    )

    dp = deg(alpha, row_l, z128)
    bp = deg(alpha, col_l, z128)

    dinv, binv = _tc_call(_inv_body, (
        jax.ShapeDtypeStruct((NPAD, FF), _f32),
        jax.ShapeDtypeStruct((NPAD, FF), _f32),
    ))(dp, bp)

    stage = pl.kernel(
        _stage_body,
        out_type=jax.ShapeDtypeStruct((NC, ACCR, FF), _f32),
        mesh=mesh,
        scratch_types=[
            pltpu.VMEM((CW,), jnp.int32),
            pltpu.VMEM((CW,), jnp.int32),
            pltpu.VMEM((CW,), jnp.int32),
            pltpu.VMEM((CW,), jnp.int32),
            pltpu.VMEM((CHA,), _f32),
            pltpu.VMEM((CHA,), _f32),
            pltpu.VMEM((CW, FF), _f32),
            pltpu.VMEM((CW, FF), _f32),
            pltpu.VMEM_SHARED((ACCR, FF), _f32),
            pltpu.SemaphoreType.DMA,
            pltpu.SemaphoreType.DMA,
        ],
    )

    combine = _tc_call(_combine_binv_body,
                       jax.ShapeDtypeStruct((NPAD, FF), _f32))

    mp = stage(hw1, alpha, row, col_l, z128)
    m1 = combine(mp, binv)
    op = stage(m1, alpha, col, row_l, z128)
    hw2 = _tc_call(_mid_body, jax.ShapeDtypeStruct((NPAD, FF), _f32))(
        op, dinv, b1t, w2b, bw2b)

    mp2 = stage(hw2, alpha, row, col_l, z128)
    m2 = combine(mp2, binv)
    op2 = stage(m2, alpha, col, row_l, z128)
    out_pad = _tc_call(_final_body, jax.ShapeDtypeStruct((NPAD, FF), _f32))(
        op2, dinv, b2t)

    return out_pad[:NN]


# speedup vs baseline: 1.1943x; 1.1943x over previous
import jax
import jax.numpy as jnp
from jax import lax
from jax.experimental import pallas as pl
from jax.experimental.pallas import tpu as pltpu
from jax.experimental.pallas import tpu_sc as plsc

NN = 10000
NE = 10000
KK = 160000
FF = 128
HID = 32
DD = 4

NC = 2
NS = 16
NWORK = NC * NS
NPAD = 10112
HNP = NPAD // 2
ACCR = 5120
RPTA = ACCR // NS
KW = KK // NWORK
CH = 125
CHA = CH * 16
CW = 128
NCHUNK = KW // CH
NCHALL = NWORK * NCHUNK
NCHSC = NCHALL // NS

_f32 = jnp.float32


def _dense_pre_body(x_ref, he_ref, wlin_ref, blin_ref, w1b_ref, bw1b_ref,
                    wa_ref, wb_ref, bs_ref, hw1_ref, a128_ref, b128_ref):
    h = jnp.dot(x_ref[...], wlin_ref[...], preferred_element_type=_f32)
    h = h + blin_ref[...]
    hw1_ref[...] = jnp.dot(h, w1b_ref[...], preferred_element_type=_f32) + bw1b_ref[...]
    a128_ref[...] = jnp.dot(h, wa_ref[...], preferred_element_type=_f32) + bs_ref[...]
    heh = jnp.dot(he_ref[...], wlin_ref[...], preferred_element_type=_f32)
    heh = heh + blin_ref[...]
    b128_ref[...] = jnp.dot(heh, wb_ref[...], preferred_element_type=_f32)


def _inv_body(dp_ref, bp_ref, dinv_ref, binv_ref):
    for c in (0, 1):
        sl = slice(c * HNP, (c + 1) * HNP)
        ds = dp_ref[c, 0:HNP]
        dinv_ref[sl, :] = jnp.where(ds > 0.0, 1.0 / ds, 0.0)
        bs = bp_ref[c, 0:HNP]
        binv_ref[sl, :] = jnp.where(bs > 0.0, 1.0 / bs, 0.0)


def _combine_binv_body(mp_ref, binv_ref, m_ref):
    for c in (0, 1):
        sl = slice(c * HNP, (c + 1) * HNP)
        m_ref[sl, :] = mp_ref[c, 0:HNP] * binv_ref[sl, :]


def _mid_body(op_ref, dinv_ref, b1_ref, w2b_ref, bw2b_ref, hw2_ref):
    o = jnp.concatenate([op_ref[0, 0:HNP], op_ref[1, 0:HNP]], axis=0)
    z = o * dinv_ref[...] + b1_ref[...]
    h2 = jnp.where(z > 0.0, z, jnp.exp(z) - 1.0)
    hw2_ref[...] = jnp.dot(h2, w2b_ref[...], preferred_element_type=_f32) + bw2b_ref[...]


def _final_body(op_ref, dinv_ref, b2_ref, out_ref):
    o = jnp.concatenate([op_ref[0, 0:HNP], op_ref[1, 0:HNP]], axis=0)
    z = o * dinv_ref[...] + b2_ref[...]
    out_ref[...] = jnp.where(z > 0.0, z, jnp.exp(z) - 1.0)


def _expand_alpha(al_v, i):
    s = al_v[pl.ds(i * 16, 16)]
    out = []
    for d in range(DD):
        scale = s.at[jnp.full((16,), d, jnp.int32)].get(mode="promise_in_bounds")
        out.append(scale)
    return out


UNROLL = 5


def _scale_mul_chunk(buf_b, al_b):
    def body(t, c):
        for u in range(UNROLL):
            i = t * UNROLL + u
            sc = _expand_alpha(al_b, i)
            for d in range(DD):
                lo = d * HID
                buf_b[i, lo:lo + 16] = buf_b[i, lo:lo + 16] * sc[d]
                buf_b[i, lo + 16:lo + 32] = buf_b[i, lo + 16:lo + 32] * sc[d]
        return c
    lax.fori_loop(0, CH // UNROLL, body, 0)


def _scale_set_chunk(buf_b, al_b):
    def body(i, c):
        sc = _expand_alpha(al_b, i)
        for d in range(DD):
            lo = d * HID
            buf_b[i, lo:lo + 16] = buf_b[i, lo:lo + 16] * 0.0 + sc[d]
            buf_b[i, lo + 16:lo + 32] = buf_b[i, lo + 16:lo + 32] * 0.0 + sc[d]
        return c
    lax.fori_loop(0, CH, body, 0)


def _alpha_body(a128_hbm, b128_hbm, row_hbm, col_hbm,
                alpha_hbm,
                ridx0_v, ridx1_v, cidx0_v, cidx1_v, arow0_v, arow1_v,
                brow0_v, brow1_v, al_v, sema0, sema1, semb0, semb1):
    cid = lax.axis_index("c")
    sid = lax.axis_index("s")
    w = sid * NC + cid
    ridx = (ridx0_v, ridx1_v)
    cidx = (cidx0_v, cidx1_v)
    arow = (arow0_v, arow1_v)
    brow = (brow0_v, brow1_v)
    semas = (sema0, sema1)
    sembs = (semb0, semb1)

    def fetch(j, b):
        r = w * NCHUNK + j
        pltpu.sync_copy(row_hbm.at[r], ridx[b])
        pltpu.sync_copy(col_hbm.at[r], cidx[b])
        pltpu.async_copy(a128_hbm.at[ridx[b]], arow[b], semas[b])
        pltpu.async_copy(b128_hbm.at[cidx[b]], brow[b], sembs[b])

    fetch(0, 0)

    def pair(t, carry):
        for b in (0, 1):
            j = t * 2 + b

            @pl.when(j + 1 < NCHUNK)
            def _(j=j, b=b):
                fetch(j + 1, 1 - b)

            pltpu.make_async_copy(a128_hbm.at[ridx[b]], arow[b], semas[b]).wait()
            pltpu.make_async_copy(b128_hbm.at[cidx[b]], brow[b], sembs[b]).wait()
            ar = arow[b]
            br = brow[b]

            def edge(t2, c2, ar=ar, br=br):
                for u in range(UNROLL):
                    i = t2 * UNROLL + u
                    v = ar[i, 0:16] + br[i, 0:16]
                    al_v[pl.ds(i * 16, 16)] = 1.0 / (1.0 + jnp.exp(-v))
                return c2

            lax.fori_loop(0, CH // UNROLL, edge, 0)
            r = w * NCHUNK + j
            pltpu.sync_copy(al_v, alpha_hbm.at[pl.ds(r * CHA, CHA)])
        return carry

    lax.fori_loop(0, NCHUNK // 2, pair, 0)


def _deg_body(alpha_hbm, sidx_hbm, z128_hbm, out_hbm,
              idx_v, al_v, buf_v, acc_s, sem):
    del sem
    cid = lax.axis_index("c")
    sid = lax.axis_index("s")
    pltpu.sync_copy(z128_hbm.at[pl.ds(sid * RPTA, RPTA)],
                    acc_s.at[pl.ds(sid * RPTA, RPTA)])
    pltpu.sync_copy(z128_hbm.at[pl.ds(0, CW)], buf_v)
    plsc.subcore_barrier()

    def chunk(j, carry):
        r = sid * NCHSC + j
        pltpu.sync_copy(sidx_hbm.at[cid, r], idx_v)
        pltpu.sync_copy(alpha_hbm.at[pl.ds(r * CHA, CHA)], al_v)

        _scale_set_chunk(buf_v, al_v)
        pltpu.sync_copy(buf_v, acc_s.at[idx_v], add=True)
        return carry

    lax.fori_loop(0, NCHSC, chunk, 0)
    plsc.subcore_barrier()
    pltpu.sync_copy(acc_s.at[pl.ds(sid * RPTA, RPTA)],
                    out_hbm.at[cid, pl.ds(sid * RPTA, RPTA)])


def _stage_body(table_hbm, alpha_hbm, gidx_hbm, sidx_hbm, z128_hbm,
                out_hbm,
                gidx0_v, gidx1_v, sidx0_v, sidx1_v, al0_v, al1_v,
                buf0_v, buf1_v, acc_s, sem0, sem1, sems0, sems1):
    cid = lax.axis_index("c")
    sid = lax.axis_index("s")
    pltpu.sync_copy(z128_hbm.at[pl.ds(sid * RPTA, RPTA)],
                    acc_s.at[pl.ds(sid * RPTA, RPTA)])
    plsc.subcore_barrier()
    gidx = (gidx0_v, gidx1_v)
    sidx = (sidx0_v, sidx1_v)
    al = (al0_v, al1_v)
    buf = (buf0_v, buf1_v)
    sems = (sem0, sem1)
    semss = (sems0, sems1)

    def fetch(j, b):
        r = sid * NCHSC + j
        pltpu.sync_copy(gidx_hbm.at[r], gidx[b])
        pltpu.sync_copy(sidx_hbm.at[cid, r], sidx[b])
        pltpu.sync_copy(alpha_hbm.at[pl.ds(r * CHA, CHA)], al[b])
        pltpu.async_copy(table_hbm.at[gidx[b]], buf[b], sems[b])

    fetch(0, 0)

    def pair(t, carry):
        for b in (0, 1):
            j = t * 2 + b
            nb = 1 - b

            pltpu.make_async_copy(table_hbm.at[gidx[b]], buf[b], sems[b]).wait()
            _scale_mul_chunk(buf[b], al[b])

            @pl.when(j > 0)
            def _(nb=nb):
                pltpu.make_async_copy(buf[nb], acc_s.at[sidx[nb]],
                                      semss[nb]).wait()

            @pl.when(j + 1 < NCHSC)
            def _(j=j, nb=nb):
                fetch(j + 1, nb)

            pltpu.async_copy(buf[b], acc_s.at[sidx[b]], semss[b], add=True)
        return carry

    lax.fori_loop(0, NCHSC // 2, pair, 0)
    pltpu.make_async_copy(buf[1], acc_s.at[sidx[1]], semss[1]).wait()
    plsc.subcore_barrier()
    pltpu.sync_copy(acc_s.at[pl.ds(sid * RPTA, RPTA)],
                    out_hbm.at[cid, pl.ds(sid * RPTA, RPTA)])


def _tc_call(body, out_shapes):
    return pl.pallas_call(body, out_shape=out_shapes)


def kernel(x, edge_index, node_types, hyperedge_types, hyperedge_attr,
           W_lin, b_lin, W_sheaf, b_sheaf, W1, bW1, bias1, W2, bW2, bias2):
    del node_types, hyperedge_types

    x_pad = jnp.pad(x, ((0, NPAD - NN), (0, 0)))
    he_pad = jnp.pad(hyperedge_attr, ((0, NPAD - NE), (0, 0)))
    pad_idx = NN + (jnp.arange(NCHALL * (CW - CH), dtype=jnp.int32)
                    .reshape(NCHALL, CW - CH) % (NPAD - NN))
    row = jnp.concatenate(
        [edge_index[0].astype(jnp.int32).reshape(NCHALL, CH), pad_idx], axis=1)
    col = jnp.concatenate(
        [edge_index[1].astype(jnp.int32).reshape(NCHALL, CH), pad_idx], axis=1)

    trash = HNP + (jnp.arange(CW, dtype=jnp.int32)[None, :] % (ACCR - HNP))

    def _split(g):
        l0 = jnp.where(g < HNP, g, trash)
        l1 = jnp.where(g >= HNP, g - HNP, trash)
        return jnp.stack([l0, l1])

    row_l = _split(row)
    col_l = _split(col)

    eyeh = jnp.eye(HID, dtype=_f32)
    m_mean = jnp.tile(eyeh, (DD, 1)) / DD
    wa = jnp.pad(m_mean @ W_sheaf[:HID], ((0, 0), (0, FF - DD)))
    wb = jnp.pad(m_mean @ W_sheaf[HID:], ((0, 0), (0, FF - DD)))
    bs128 = jnp.pad(b_sheaf, (0, FF - DD)).reshape(1, FF)
    w1b = jnp.kron(jnp.eye(DD, dtype=_f32), W1)
    w2b = jnp.kron(jnp.eye(DD, dtype=_f32), W2)
    bw1b = jnp.tile(bW1, DD).reshape(1, FF)
    bw2b = jnp.tile(bW2, DD).reshape(1, FF)
    b1t = jnp.tile(bias1, DD).reshape(1, FF)
    b2t = jnp.tile(bias2, DD).reshape(1, FF)
    z128 = jnp.zeros((NPAD, FF), _f32)

    hw1, a128, b128 = _tc_call(_dense_pre_body, (
        jax.ShapeDtypeStruct((NPAD, FF), _f32),
        jax.ShapeDtypeStruct((NPAD, FF), _f32),
        jax.ShapeDtypeStruct((NPAD, FF), _f32),
    ))(x_pad, he_pad, W_lin, b_lin.reshape(1, FF), w1b, bw1b, wa, wb, bs128)

    mesh = plsc.VectorSubcoreMesh(core_axis_name="c", subcore_axis_name="s")

    alpha = pl.kernel(
        _alpha_body,
        out_type=jax.ShapeDtypeStruct((NCHALL * CHA,), _f32),
        mesh=mesh,
        scratch_types=[
            pltpu.VMEM((CW,), jnp.int32),
            pltpu.VMEM((CW,), jnp.int32),
            pltpu.VMEM((CW,), jnp.int32),
            pltpu.VMEM((CW,), jnp.int32),
            pltpu.VMEM((CW, FF), _f32),
            pltpu.VMEM((CW, FF), _f32),
            pltpu.VMEM((CW, FF), _f32),
            pltpu.VMEM((CW, FF), _f32),
            pltpu.VMEM((CHA,), _f32),
            pltpu.SemaphoreType.DMA,
            pltpu.SemaphoreType.DMA,
            pltpu.SemaphoreType.DMA,
            pltpu.SemaphoreType.DMA,
        ],
    )(a128, b128, row, col)

    deg = pl.kernel(
        _deg_body,
        out_type=jax.ShapeDtypeStruct((NC, ACCR, FF), _f32),
        mesh=mesh,
        scratch_types=[
            pltpu.VMEM((CW,), jnp.int32),
            pltpu.VMEM((CHA,), _f32),
            pltpu.VMEM((CW, FF), _f32),
            pltpu.VMEM_SHARED((ACCR, FF), _f32),
            pltpu.SemaphoreType.DMA,
        ],
    )

    dp = deg(alpha, row_l, z128)
    bp = deg(alpha, col_l, z128)

    dinv, binv = _tc_call(_inv_body, (
        jax.ShapeDtypeStruct((NPAD, FF), _f32),
        jax.ShapeDtypeStruct((NPAD, FF), _f32),
    ))(dp, bp)

    stage = pl.kernel(
        _stage_body,
        out_type=jax.ShapeDtypeStruct((NC, ACCR, FF), _f32),
        mesh=mesh,
        scratch_types=[
            pltpu.VMEM((CW,), jnp.int32),
            pltpu.VMEM((CW,), jnp.int32),
            pltpu.VMEM((CW,), jnp.int32),
            pltpu.VMEM((CW,), jnp.int32),
            pltpu.VMEM((CHA,), _f32),
            pltpu.VMEM((CHA,), _f32),
            pltpu.VMEM((CW, FF), _f32),
            pltpu.VMEM((CW, FF), _f32),
            pltpu.VMEM_SHARED((ACCR, FF), _f32),
            pltpu.SemaphoreType.DMA,
            pltpu.SemaphoreType.DMA,
            pltpu.SemaphoreType.DMA,
            pltpu.SemaphoreType.DMA,
        ],
    )

    combine = _tc_call(_combine_binv_body,
                       jax.ShapeDtypeStruct((NPAD, FF), _f32))

    mp = stage(hw1, alpha, row, col_l, z128)
    m1 = combine(mp, binv)
    op = stage(m1, alpha, col, row_l, z128)
    hw2 = _tc_call(_mid_body, jax.ShapeDtypeStruct((NPAD, FF), _f32))(
        op, dinv, b1t, w2b, bw2b)

    mp2 = stage(hw2, alpha, row, col_l, z128)
    m2 = combine(mp2, binv)
    op2 = stage(m2, alpha, col, row_l, z128)
    out_pad = _tc_call(_final_body, jax.ShapeDtypeStruct((NPAD, FF), _f32))(
        op2, dinv, b2t)

    return out_pad[:NN]


# speedup vs baseline: 1.1985x; 1.0035x over previous
import jax
import jax.numpy as jnp
from jax import lax
from jax.experimental import pallas as pl
from jax.experimental.pallas import tpu as pltpu
from jax.experimental.pallas import tpu_sc as plsc

NN = 10000
NE = 10000
KK = 160000
FF = 128
HID = 32
DD = 4

NC = 2
NS = 16
NWORK = NC * NS
NPAD = 10112
HNP = NPAD // 2
ACCR = 5120
RPTA = ACCR // NS
KW = KK // NWORK
CH = 125
CHA = CH * 16
CW = 128
NCHUNK = KW // CH
NCHALL = NWORK * NCHUNK
NCHSC = NCHALL // NS

_f32 = jnp.float32


def _dense_pre_body(x_ref, he_ref, wlin_ref, blin_ref, w1b_ref, bw1b_ref,
                    wa_ref, wb_ref, bs_ref, hw1_ref, a128_ref, b128_ref):
    h = jnp.dot(x_ref[...], wlin_ref[...], preferred_element_type=_f32)
    h = h + blin_ref[...]
    hw1_ref[...] = jnp.dot(h, w1b_ref[...], preferred_element_type=_f32) + bw1b_ref[...]
    a128_ref[...] = jnp.dot(h, wa_ref[...], preferred_element_type=_f32) + bs_ref[...]
    heh = jnp.dot(he_ref[...], wlin_ref[...], preferred_element_type=_f32)
    heh = heh + blin_ref[...]
    b128_ref[...] = jnp.dot(heh, wb_ref[...], preferred_element_type=_f32)


def _inv_body(dp_ref, bp_ref, dinv_ref, binv_ref):
    for c in (0, 1):
        sl = slice(c * HNP, (c + 1) * HNP)
        ds = dp_ref[c, 0:HNP]
        dinv_ref[sl, :] = jnp.where(ds > 0.0, 1.0 / ds, 0.0)
        bs = bp_ref[c, 0:HNP]
        binv_ref[sl, :] = jnp.where(bs > 0.0, 1.0 / bs, 0.0)


def _combine_binv_body(mp_ref, binv_ref, m_ref):
    for c in (0, 1):
        sl = slice(c * HNP, (c + 1) * HNP)
        m_ref[sl, :] = mp_ref[c, 0:HNP] * binv_ref[sl, :]


def _mid_body(op_ref, dinv_ref, b1_ref, w2b_ref, bw2b_ref, hw2_ref):
    o = jnp.concatenate([op_ref[0, 0:HNP], op_ref[1, 0:HNP]], axis=0)
    z = o * dinv_ref[...] + b1_ref[...]
    h2 = jnp.where(z > 0.0, z, jnp.exp(z) - 1.0)
    hw2_ref[...] = jnp.dot(h2, w2b_ref[...], preferred_element_type=_f32) + bw2b_ref[...]


def _final_body(op_ref, dinv_ref, b2_ref, out_ref):
    o = jnp.concatenate([op_ref[0, 0:HNP], op_ref[1, 0:HNP]], axis=0)
    z = o * dinv_ref[...] + b2_ref[...]
    out_ref[...] = jnp.where(z > 0.0, z, jnp.exp(z) - 1.0)


def _expand_alpha(al_v, i):
    s = al_v[pl.ds(i * 16, 16)]
    out = []
    for d in range(DD):
        scale = s.at[jnp.full((16,), d, jnp.int32)].get(mode="promise_in_bounds")
        out.append(scale)
    return out


UNROLL = 1


def _scale_mul_chunk(buf_b, al_b):
    def body(t, c):
        for u in range(UNROLL):
            i = t * UNROLL + u
            sc = _expand_alpha(al_b, i)
            for d in range(DD):
                lo = d * HID
                buf_b[i, lo:lo + 16] = buf_b[i, lo:lo + 16] * sc[d]
                buf_b[i, lo + 16:lo + 32] = buf_b[i, lo + 16:lo + 32] * sc[d]
        return c
    lax.fori_loop(0, CH // UNROLL, body, 0)


def _scale_set_chunk(buf_b, al_b):
    def body(i, c):
        sc = _expand_alpha(al_b, i)
        for d in range(DD):
            lo = d * HID
            buf_b[i, lo:lo + 16] = buf_b[i, lo:lo + 16] * 0.0 + sc[d]
            buf_b[i, lo + 16:lo + 32] = buf_b[i, lo + 16:lo + 32] * 0.0 + sc[d]
        return c
    lax.fori_loop(0, CH, body, 0)


def _alpha_body(a128_hbm, b128_hbm, row_hbm, col_hbm,
                alpha_hbm,
                ridx0_v, ridx1_v, cidx0_v, cidx1_v, arow0_v, arow1_v,
                brow0_v, brow1_v, al_v, sema0, sema1, semb0, semb1):
    cid = lax.axis_index("c")
    sid = lax.axis_index("s")
    w = sid * NC + cid
    ridx = (ridx0_v, ridx1_v)
    cidx = (cidx0_v, cidx1_v)
    arow = (arow0_v, arow1_v)
    brow = (brow0_v, brow1_v)
    semas = (sema0, sema1)
    sembs = (semb0, semb1)

    def fetch(j, b):
        r = w * NCHUNK + j
        pltpu.sync_copy(row_hbm.at[r], ridx[b])
        pltpu.sync_copy(col_hbm.at[r], cidx[b])
        pltpu.async_copy(a128_hbm.at[ridx[b]], arow[b], semas[b])
        pltpu.async_copy(b128_hbm.at[cidx[b]], brow[b], sembs[b])

    fetch(0, 0)

    def pair(t, carry):
        for b in (0, 1):
            j = t * 2 + b

            @pl.when(j + 1 < NCHUNK)
            def _(j=j, b=b):
                fetch(j + 1, 1 - b)

            pltpu.make_async_copy(a128_hbm.at[ridx[b]], arow[b], semas[b]).wait()
            pltpu.make_async_copy(b128_hbm.at[cidx[b]], brow[b], sembs[b]).wait()
            ar = arow[b]
            br = brow[b]

            def edge(t2, c2, ar=ar, br=br):
                for u in range(UNROLL):
                    i = t2 * UNROLL + u
                    v = ar[i, 0:16] + br[i, 0:16]
                    al_v[pl.ds(i * 16, 16)] = 1.0 / (1.0 + jnp.exp(-v))
                return c2

            lax.fori_loop(0, CH // UNROLL, edge, 0)
            r = w * NCHUNK + j
            pltpu.sync_copy(al_v, alpha_hbm.at[pl.ds(r * CHA, CHA)])
        return carry

    lax.fori_loop(0, NCHUNK // 2, pair, 0)


def _deg_body(alpha_hbm, sidx_hbm, z128_hbm, out_hbm,
              idx_v, al_v, buf_v, acc_s, sem):
    del sem
    cid = lax.axis_index("c")
    sid = lax.axis_index("s")
    pltpu.sync_copy(z128_hbm.at[pl.ds(sid * RPTA, RPTA)],
                    acc_s.at[pl.ds(sid * RPTA, RPTA)])
    pltpu.sync_copy(z128_hbm.at[pl.ds(0, CW)], buf_v)
    plsc.subcore_barrier()

    def chunk(j, carry):
        r = sid * NCHSC + j
        pltpu.sync_copy(sidx_hbm.at[cid, r], idx_v)
        pltpu.sync_copy(alpha_hbm.at[pl.ds(r * CHA, CHA)], al_v)

        _scale_set_chunk(buf_v, al_v)
        pltpu.sync_copy(buf_v, acc_s.at[idx_v], add=True)
        return carry

    lax.fori_loop(0, NCHSC, chunk, 0)
    plsc.subcore_barrier()
    pltpu.sync_copy(acc_s.at[pl.ds(sid * RPTA, RPTA)],
                    out_hbm.at[cid, pl.ds(sid * RPTA, RPTA)])


def _stage_body(table_hbm, alpha_hbm, gidx_hbm, sidx_hbm, z128_hbm,
                out_hbm,
                gidx0_v, gidx1_v, sidx0_v, sidx1_v, al0_v, al1_v,
                buf0_v, buf1_v, acc_s, sem0, sem1, sems0, sems1):
    cid = lax.axis_index("c")
    sid = lax.axis_index("s")
    pltpu.sync_copy(z128_hbm.at[pl.ds(sid * RPTA, RPTA)],
                    acc_s.at[pl.ds(sid * RPTA, RPTA)])
    plsc.subcore_barrier()
    gidx = (gidx0_v, gidx1_v)
    sidx = (sidx0_v, sidx1_v)
    al = (al0_v, al1_v)
    buf = (buf0_v, buf1_v)
    sems = (sem0, sem1)
    semss = (sems0, sems1)

    def fetch(j, b):
        r = sid * NCHSC + j
        pltpu.sync_copy(gidx_hbm.at[r], gidx[b])
        pltpu.sync_copy(sidx_hbm.at[cid, r], sidx[b])
        pltpu.sync_copy(alpha_hbm.at[pl.ds(r * CHA, CHA)], al[b])
        pltpu.async_copy(table_hbm.at[gidx[b]], buf[b], sems[b])

    fetch(0, 0)

    def pair(t, carry):
        for b in (0, 1):
            j = t * 2 + b
            nb = 1 - b

            pltpu.make_async_copy(table_hbm.at[gidx[b]], buf[b], sems[b]).wait()
            _scale_mul_chunk(buf[b], al[b])

            @pl.when(j > 0)
            def _(nb=nb):
                pltpu.make_async_copy(buf[nb], acc_s.at[sidx[nb]],
                                      semss[nb]).wait()

            @pl.when(j + 1 < NCHSC)
            def _(j=j, nb=nb):
                fetch(j + 1, nb)

            pltpu.async_copy(buf[b], acc_s.at[sidx[b]], semss[b], add=True)
        return carry

    lax.fori_loop(0, NCHSC // 2, pair, 0)
    pltpu.make_async_copy(buf[1], acc_s.at[sidx[1]], semss[1]).wait()
    plsc.subcore_barrier()
    pltpu.sync_copy(acc_s.at[pl.ds(sid * RPTA, RPTA)],
                    out_hbm.at[cid, pl.ds(sid * RPTA, RPTA)])


def _tc_call(body, out_shapes):
    return pl.pallas_call(body, out_shape=out_shapes)


def kernel(x, edge_index, node_types, hyperedge_types, hyperedge_attr,
           W_lin, b_lin, W_sheaf, b_sheaf, W1, bW1, bias1, W2, bW2, bias2):
    del node_types, hyperedge_types

    x_pad = jnp.pad(x, ((0, NPAD - NN), (0, 0)))
    he_pad = jnp.pad(hyperedge_attr, ((0, NPAD - NE), (0, 0)))
    pad_idx = NN + (jnp.arange(NCHALL * (CW - CH), dtype=jnp.int32)
                    .reshape(NCHALL, CW - CH) % (NPAD - NN))
    row = jnp.concatenate(
        [edge_index[0].astype(jnp.int32).reshape(NCHALL, CH), pad_idx], axis=1)
    col = jnp.concatenate(
        [edge_index[1].astype(jnp.int32).reshape(NCHALL, CH), pad_idx], axis=1)

    trash = HNP + (jnp.arange(CW, dtype=jnp.int32)[None, :] % (ACCR - HNP))

    def _split(g):
        l0 = jnp.where(g < HNP, g, trash)
        l1 = jnp.where(g >= HNP, g - HNP, trash)
        return jnp.stack([l0, l1])

    row_l = _split(row)
    col_l = _split(col)

    eyeh = jnp.eye(HID, dtype=_f32)
    m_mean = jnp.tile(eyeh, (DD, 1)) / DD
    wa = jnp.pad(m_mean @ W_sheaf[:HID], ((0, 0), (0, FF - DD)))
    wb = jnp.pad(m_mean @ W_sheaf[HID:], ((0, 0), (0, FF - DD)))
    bs128 = jnp.pad(b_sheaf, (0, FF - DD)).reshape(1, FF)
    w1b = jnp.kron(jnp.eye(DD, dtype=_f32), W1)
    w2b = jnp.kron(jnp.eye(DD, dtype=_f32), W2)
    bw1b = jnp.tile(bW1, DD).reshape(1, FF)
    bw2b = jnp.tile(bW2, DD).reshape(1, FF)
    b1t = jnp.tile(bias1, DD).reshape(1, FF)
    b2t = jnp.tile(bias2, DD).reshape(1, FF)
    z128 = jnp.zeros((NPAD, FF), _f32)

    hw1, a128, b128 = _tc_call(_dense_pre_body, (
        jax.ShapeDtypeStruct((NPAD, FF), _f32),
        jax.ShapeDtypeStruct((NPAD, FF), _f32),
        jax.ShapeDtypeStruct((NPAD, FF), _f32),
    ))(x_pad, he_pad, W_lin, b_lin.reshape(1, FF), w1b, bw1b, wa, wb, bs128)

    mesh = plsc.VectorSubcoreMesh(core_axis_name="c", subcore_axis_name="s")

    alpha = pl.kernel(
        _alpha_body,
        out_type=jax.ShapeDtypeStruct((NCHALL * CHA,), _f32),
        mesh=mesh,
        scratch_types=[
            pltpu.VMEM((CW,), jnp.int32),
            pltpu.VMEM((CW,), jnp.int32),
            pltpu.VMEM((CW,), jnp.int32),
            pltpu.VMEM((CW,), jnp.int32),
            pltpu.VMEM((CW, FF), _f32),
            pltpu.VMEM((CW, FF), _f32),
            pltpu.VMEM((CW, FF), _f32),
            pltpu.VMEM((CW, FF), _f32),
            pltpu.VMEM((CHA,), _f32),
            pltpu.SemaphoreType.DMA,
            pltpu.SemaphoreType.DMA,
            pltpu.SemaphoreType.DMA,
            pltpu.SemaphoreType.DMA,
        ],
    )(a128, b128, row, col)

    deg = pl.kernel(
        _deg_body,
        out_type=jax.ShapeDtypeStruct((NC, ACCR, FF), _f32),
        mesh=mesh,
        scratch_types=[
            pltpu.VMEM((CW,), jnp.int32),
            pltpu.VMEM((CHA,), _f32),
            pltpu.VMEM((CW, FF), _f32),
            pltpu.VMEM_SHARED((ACCR, FF), _f32),
            pltpu.SemaphoreType.DMA,
        ],
    )

    dp = deg(alpha, row_l, z128)
    bp = deg(alpha, col_l, z128)

    dinv, binv = _tc_call(_inv_body, (
        jax.ShapeDtypeStruct((NPAD, FF), _f32),
        jax.ShapeDtypeStruct((NPAD, FF), _f32),
    ))(dp, bp)

    stage = pl.kernel(
        _stage_body,
        out_type=jax.ShapeDtypeStruct((NC, ACCR, FF), _f32),
        mesh=mesh,
        scratch_types=[
            pltpu.VMEM((CW,), jnp.int32),
            pltpu.VMEM((CW,), jnp.int32),
            pltpu.VMEM((CW,), jnp.int32),
            pltpu.VMEM((CW,), jnp.int32),
            pltpu.VMEM((CHA,), _f32),
            pltpu.VMEM((CHA,), _f32),
            pltpu.VMEM((CW, FF), _f32),
            pltpu.VMEM((CW, FF), _f32),
            pltpu.VMEM_SHARED((ACCR, FF), _f32),
            pltpu.SemaphoreType.DMA,
            pltpu.SemaphoreType.DMA,
            pltpu.SemaphoreType.DMA,
            pltpu.SemaphoreType.DMA,
        ],
    )

    combine = _tc_call(_combine_binv_body,
                       jax.ShapeDtypeStruct((NPAD, FF), _f32))

    mp = stage(hw1, alpha, row, col_l, z128)
    m1 = combine(mp, binv)
    op = stage(m1, alpha, col, row_l, z128)
    hw2 = _tc_call(_mid_body, jax.ShapeDtypeStruct((NPAD, FF), _f32))(
        op, dinv, b1t, w2b, bw2b)

    mp2 = stage(hw2, alpha, row, col_l, z128)
    m2 = combine(mp2, binv)
    op2 = stage(m2, alpha, col, row_l, z128)
    out_pad = _tc_call(_final_body, jax.ShapeDtypeStruct((NPAD, FF), _f32))(
        op2, dinv, b2t)

    return out_pad[:NN]
